# baseline (speedup 1.0000x reference)
import functools

import jax
import jax.numpy as jnp
from jax import lax
from jax.experimental import pallas as pl
from jax.experimental.pallas import tpu as pltpu
from jax.experimental.pallas import tpu_sc as plsc

_N, _E, _D = 10000, 320000, 128
_NC, _NS = 2, 16
_HD = _D // _NC
_EPT = _E // _NS
_C = 125
_CH = _EPT // _C
_NB = 4
_RPT = 624
_RC = 104
_RCH = _RPT // _RC
_TAIL0 = _NS * _RPT
_TAILN = _N - _TAIL0


def _sc_aggregate(x_split, src_t, dst_t):
  mesh = plsc.VectorSubcoreMesh(core_axis_name="c", subcore_axis_name="s")

  @functools.partial(
      pl.kernel,
      out_type=jax.ShapeDtypeStruct((_NC, _N, _HD), jnp.float32),
      mesh=mesh,
      compiler_params=pltpu.CompilerParams(use_tc_tiling_on_sc=False),
      scratch_types=[
          pltpu.VMEM((_CH, _C), jnp.int32),
          pltpu.VMEM((_CH, _C), jnp.int32),
          pltpu.VMEM((_NB, _C, _HD), jnp.float32),
          pltpu.VMEM_SHARED((_N, _HD), jnp.float32),
          pltpu.SemaphoreType.DMA((_NB,)),
          pltpu.SemaphoreType.DMA((_NB,)),
      ],
  )
  def agg_kernel(x_hbm, src_hbm, dst_hbm, out_hbm, src_v, dst_v, rows_v,
                 agg_sh, gsem, ssem):
    c = lax.axis_index("c")
    s = lax.axis_index("s")
    xc = x_hbm.at[c]

    pltpu.sync_copy(src_hbm.at[s], src_v)
    pltpu.sync_copy(dst_hbm.at[s], dst_v)

    r0 = s * _RPT
    pltpu.sync_copy(xc.at[pl.ds(r0, _RPT)], agg_sh.at[pl.ds(r0, _RPT)])

    @pl.when(s == _NS - 1)
    def _():
      pltpu.sync_copy(xc.at[pl.ds(_TAIL0, _TAILN)],
                      agg_sh.at[pl.ds(_TAIL0, _TAILN)])

    plsc.subcore_barrier()

    for b in range(_NB):
      pltpu.async_copy(xc.at[src_v.at[b]], rows_v.at[b], gsem.at[b])

    @pl.loop(0, _CH, step=_NB)
    def _(j0):
      for b in range(_NB):
        pltpu.make_async_copy(xc.at[src_v.at[b]], rows_v.at[b],
                              gsem.at[b]).wait()
        pltpu.async_copy(rows_v.at[b], agg_sh.at[dst_v.at[j0 + b]],
                         ssem.at[b], add=True)
      for b in range(_NB):
        nj = j0 + b + _NB
        pltpu.make_async_copy(rows_v.at[b], agg_sh.at[dst_v.at[j0 + b]],
                              ssem.at[b]).wait()

        @pl.when(nj < _CH)
        def _():
          pltpu.async_copy(xc.at[src_v.at[nj]], rows_v.at[b], gsem.at[b])

    plsc.subcore_barrier()

    pltpu.sync_copy(agg_sh.at[pl.ds(r0, _RPT)], out_hbm.at[c, pl.ds(r0, _RPT)])

    @pl.when(s == _NS - 1)
    def _():
      pltpu.sync_copy(agg_sh.at[pl.ds(_TAIL0, _TAILN)],
                      out_hbm.at[c, pl.ds(_TAIL0, _TAILN)])

  return agg_kernel(x_split, src_t, dst_t)


def _mlp_bn(agg, W1, b1, W2, b2, g, be, split_out):
  def body(a, W1r, b1r, W2r, b2r, gr, ber, out):
    z = jnp.dot(a[0], W1r[:_HD, :]) + jnp.dot(a[1], W1r[_HD:, :])
    z = jnp.maximum(z + b1r[...], 0.0)
    z = jnp.dot(z, W2r[...]) + b2r[...]
    mu = jnp.mean(z, axis=0, keepdims=True)
    var = jnp.mean((z - mu) * (z - mu), axis=0, keepdims=True)
    zn = (z - mu) * lax.rsqrt(var + 1e-5) * gr[...] + ber[...]
    zn = jnp.maximum(zn, 0.0)
    if split_out:
      out[0] = zn[:, :_HD]
      out[1] = zn[:, _HD:]
    else:
      out[...] = zn

  out_shape = (jax.ShapeDtypeStruct((_NC, _N, _HD), jnp.float32)
               if split_out else jax.ShapeDtypeStruct((_N, _D), jnp.float32))
  return pl.pallas_call(body, out_shape=out_shape)(
      agg, W1, b1.reshape(1, _D), W2, b2.reshape(1, _D),
      g.reshape(1, _D), be.reshape(1, _D))


def kernel(x, edge_index, W1_0, b1_0, W2_0, b2_0, g_0, be_0, W1_1, b1_1,
           W2_1, b2_1, g_1, be_1, W1_2, b1_2, W2_2, b2_2, g_2, be_2):
  x = x.astype(jnp.bfloat16).astype(jnp.float32)
  src_t = edge_index[0].reshape(_NS, _CH, _C)
  dst_t = edge_index[1].reshape(_NS, _CH, _C)
  params = [(W1_0, b1_0, W2_0, b2_0, g_0, be_0),
            (W1_1, b1_1, W2_1, b2_1, g_1, be_1),
            (W1_2, b1_2, W2_2, b2_2, g_2, be_2)]
  xs = jnp.stack([x[:, :_HD], x[:, _HD:]])
  for l, (W1, b1, W2, b2, g, be) in enumerate(params):
    agg = _sc_aggregate(xs, src_t, dst_t)
    last = l == len(params) - 1
    xs = _mlp_bn(agg, W1, b1, W2, b2, g, be, split_out=not last)
  return xs


# speedup vs baseline: 1.1312x; 1.1312x over previous
import functools

import jax
import jax.numpy as jnp
from jax import lax
from jax.experimental import pallas as pl
from jax.experimental.pallas import tpu as pltpu
from jax.experimental.pallas import tpu_sc as plsc

_N, _E, _D = 10000, 320000, 128
_NC, _NS = 2, 16
_HD = _D // _NC
_EPT = _E // _NS
_C = 125
_CH = _EPT // _C
_NB = 4
_RPT = 624
_RC = 104
_RCH = _RPT // _RC
_TAIL0 = _NS * _RPT
_TAILN = _N - _TAIL0


def _sc_aggregate(x_split, src_t, dst_t):
  mesh = plsc.VectorSubcoreMesh(core_axis_name="c", subcore_axis_name="s")

  @functools.partial(
      pl.kernel,
      out_type=jax.ShapeDtypeStruct((_NC, _N, _HD), jnp.float32),
      mesh=mesh,
      compiler_params=pltpu.CompilerParams(use_tc_tiling_on_sc=False),
      scratch_types=[
          pltpu.VMEM((_CH, _C), jnp.int32),
          pltpu.VMEM((_CH, _C), jnp.int32),
          pltpu.VMEM((_NB, _C, _HD), jnp.float32),
          pltpu.VMEM_SHARED((_N, _HD), jnp.float32),
          pltpu.SemaphoreType.DMA((_NB,)),
      ],
  )
  def agg_kernel(x_hbm, src_hbm, dst_hbm, out_hbm, src_v, dst_v, rows_v,
                 agg_sh, gsem):
    c = lax.axis_index("c")
    s = lax.axis_index("s")
    xc = x_hbm.at[c]

    pltpu.sync_copy(src_hbm.at[s], src_v)
    pltpu.sync_copy(dst_hbm.at[s], dst_v)

    r0 = s * _RPT
    pltpu.sync_copy(xc.at[pl.ds(r0, _RPT)], agg_sh.at[pl.ds(r0, _RPT)])

    @pl.when(s == _NS - 1)
    def _():
      pltpu.sync_copy(xc.at[pl.ds(_TAIL0, _TAILN)],
                      agg_sh.at[pl.ds(_TAIL0, _TAILN)])

    plsc.subcore_barrier()

    for b in range(_NB):
      pltpu.async_copy(xc.at[src_v.at[b]], rows_v.at[b], gsem.at[b])

    @pl.loop(0, _CH, step=_NB)
    def _(j0):
      for b in range(_NB):
        j = j0 + b
        pltpu.make_async_copy(xc.at[src_v.at[b]], rows_v.at[b],
                              gsem.at[b]).wait()
        pltpu.sync_copy(rows_v.at[b], agg_sh.at[dst_v.at[j]], add=True)
        nj = j + _NB

        @pl.when(nj < _CH)
        def _():
          pltpu.async_copy(xc.at[src_v.at[nj]], rows_v.at[b], gsem.at[b])

    plsc.subcore_barrier()

    pltpu.sync_copy(agg_sh.at[pl.ds(r0, _RPT)], out_hbm.at[c, pl.ds(r0, _RPT)])

    @pl.when(s == _NS - 1)
    def _():
      pltpu.sync_copy(agg_sh.at[pl.ds(_TAIL0, _TAILN)],
                      out_hbm.at[c, pl.ds(_TAIL0, _TAILN)])

  return agg_kernel(x_split, src_t, dst_t)


def _mlp_bn(agg, W1, b1, W2, b2, g, be, split_out):
  def body(a, W1r, b1r, W2r, b2r, gr, ber, out):
    z = jnp.dot(a[0], W1r[:_HD, :]) + jnp.dot(a[1], W1r[_HD:, :])
    z = jnp.maximum(z + b1r[...], 0.0)
    z = jnp.dot(z, W2r[...]) + b2r[...]
    mu = jnp.mean(z, axis=0, keepdims=True)
    var = jnp.mean((z - mu) * (z - mu), axis=0, keepdims=True)
    zn = (z - mu) * lax.rsqrt(var + 1e-5) * gr[...] + ber[...]
    zn = jnp.maximum(zn, 0.0)
    if split_out:
      out[0] = zn[:, :_HD]
      out[1] = zn[:, _HD:]
    else:
      out[...] = zn

  out_shape = (jax.ShapeDtypeStruct((_NC, _N, _HD), jnp.float32)
               if split_out else jax.ShapeDtypeStruct((_N, _D), jnp.float32))
  return pl.pallas_call(body, out_shape=out_shape)(
      agg, W1, b1.reshape(1, _D), W2, b2.reshape(1, _D),
      g.reshape(1, _D), be.reshape(1, _D))


def kernel(x, edge_index, W1_0, b1_0, W2_0, b2_0, g_0, be_0, W1_1, b1_1,
           W2_1, b2_1, g_1, be_1, W1_2, b1_2, W2_2, b2_2, g_2, be_2):
  x = x.astype(jnp.bfloat16).astype(jnp.float32)
  src_t = edge_index[0].reshape(_NS, _CH, _C)
  dst_t = edge_index[1].reshape(_NS, _CH, _C)
  params = [(W1_0, b1_0, W2_0, b2_0, g_0, be_0),
            (W1_1, b1_1, W2_1, b2_1, g_1, be_1),
            (W1_2, b1_2, W2_2, b2_2, g_2, be_2)]
  xs = jnp.stack([x[:, :_HD], x[:, _HD:]])
  for l, (W1, b1, W2, b2, g, be) in enumerate(params):
    agg = _sc_aggregate(xs, src_t, dst_t)
    last = l == len(params) - 1
    xs = _mlp_bn(agg, W1, b1, W2, b2, g, be, split_out=not last)
  return xs


# speedup vs baseline: 1.3183x; 1.1653x over previous
import functools

import jax
import jax.numpy as jnp
from jax import lax
from jax.experimental import pallas as pl
from jax.experimental.pallas import tpu as pltpu
from jax.experimental.pallas import tpu_sc as plsc

_N, _E, _D = 10000, 320000, 128
_NC, _NS = 2, 16
_HD = _D // _NC
_EPT = _E // _NS
_C = 125
_CH = _EPT // _C
_NB = 4
_RPT = 624
_RC = 104
_RCH = _RPT // _RC
_TAIL0 = _NS * _RPT
_TAILN = _N - _TAIL0


def _sc_aggregate(x_split, et):
  mesh = plsc.VectorSubcoreMesh(core_axis_name="c", subcore_axis_name="s")

  @functools.partial(
      pl.kernel,
      out_type=jax.ShapeDtypeStruct((_NC, _N, _HD), jnp.float32),
      mesh=mesh,
      compiler_params=pltpu.CompilerParams(use_tc_tiling_on_sc=False),
      scratch_types=[
          pltpu.VMEM((_CH, _C), jnp.int32),
          pltpu.VMEM((_CH, _C), jnp.int32),
          pltpu.VMEM((_NB, _C, _HD), jnp.float32),
          pltpu.VMEM_SHARED((_N, _HD), jnp.float32),
          pltpu.SemaphoreType.DMA((_NB,)),
      ],
  )
  def agg_kernel(x_hbm, et_hbm, out_hbm, src_v, dst_v, rows_v,
                 agg_sh, gsem):
    c = lax.axis_index("c")
    s = lax.axis_index("s")
    xc = x_hbm.at[c]

    pltpu.sync_copy(et_hbm.at[0, s], src_v)
    pltpu.sync_copy(et_hbm.at[1, s], dst_v)

    r0 = s * _RPT
    pltpu.sync_copy(xc.at[pl.ds(r0, _RPT)], agg_sh.at[pl.ds(r0, _RPT)])

    @pl.when(s == _NS - 1)
    def _():
      pltpu.sync_copy(xc.at[pl.ds(_TAIL0, _TAILN)],
                      agg_sh.at[pl.ds(_TAIL0, _TAILN)])

    plsc.subcore_barrier()

    for b in range(_NB):
      pltpu.async_copy(xc.at[src_v.at[b]], rows_v.at[b], gsem.at[b])

    @pl.loop(0, _CH, step=_NB)
    def _(j0):
      for b in range(_NB):
        j = j0 + b
        pltpu.make_async_copy(xc.at[src_v.at[b]], rows_v.at[b],
                              gsem.at[b]).wait()
        pltpu.sync_copy(rows_v.at[b], agg_sh.at[dst_v.at[j]], add=True)
        nj = j + _NB

        @pl.when(nj < _CH)
        def _():
          pltpu.async_copy(xc.at[src_v.at[nj]], rows_v.at[b], gsem.at[b])

    plsc.subcore_barrier()

    pltpu.sync_copy(agg_sh.at[pl.ds(r0, _RPT)], out_hbm.at[c, pl.ds(r0, _RPT)])

    @pl.when(s == _NS - 1)
    def _():
      pltpu.sync_copy(agg_sh.at[pl.ds(_TAIL0, _TAILN)],
                      out_hbm.at[c, pl.ds(_TAIL0, _TAILN)])

  return agg_kernel(x_split, et)


def _mlp_bn(agg, W1, b1, W2, b2, g, be, split_out):
  if split_out:
    zero = jnp.zeros((_HD, _D), jnp.float32)
    W1a, W1b = W1[:_HD, :], W1[_HD:, :]
    B0 = jnp.concatenate([jnp.concatenate([W1a, zero], 1),
                          jnp.concatenate([zero, W1a], 1)], 0)
    B1 = jnp.concatenate([jnp.concatenate([W1b, zero], 1),
                          jnp.concatenate([zero, W1b], 1)], 0)
    zero2 = jnp.zeros((_D, _D), jnp.float32)
    W2d = jnp.concatenate([jnp.concatenate([W2, zero2], 1),
                           jnp.concatenate([zero2, W2], 1)], 0)
    b1t = jnp.concatenate([b1, b1]).reshape(1, 2 * _D)
    b2t = jnp.concatenate([b2, b2]).reshape(1, 2 * _D)
    gt = jnp.concatenate([g, g]).reshape(1, 2 * _D)
    bet = jnp.concatenate([be, be]).reshape(1, 2 * _D)

    def body(a, B0r, B1r, W2r, b1r, b2r, gr, ber, out):
      z = jnp.dot(a[0], B0r[...]) + jnp.dot(a[1], B1r[...])
      z = jnp.maximum(z + b1r[...], 0.0)
      z = jnp.dot(z, W2r[...]) + b2r[...]
      m2 = jnp.mean(z, axis=0, keepdims=True)
      mu = (m2[:, :_D] + m2[:, _D:]) * 0.5
      mut = jnp.concatenate([mu, mu], axis=1)
      d = z - mut
      v2 = jnp.mean(d * d, axis=0, keepdims=True)
      var = (v2[:, :_D] + v2[:, _D:]) * 0.5
      vart = jnp.concatenate([var, var], axis=1)
      zn = d * lax.rsqrt(vart + 1e-5) * gr[...] + ber[...]
      zn = jnp.maximum(zn, 0.0)
      out[0] = jnp.concatenate([zn[:, :_HD], zn[:, _D:_D + _HD]], axis=1)
      out[1] = jnp.concatenate([zn[:, _HD:_D], zn[:, _D + _HD:]], axis=1)

    return pl.pallas_call(
        body,
        out_shape=jax.ShapeDtypeStruct((_NC, _N // 2, _D), jnp.float32),
    )(agg.reshape(_NC, _N // 2, _D), B0, B1, W2d, b1t, b2t, gt, bet)

  def body(a, W1r, b1r, W2r, b2r, gr, ber, out):
    z = jnp.dot(a[0], W1r[:_HD, :]) + jnp.dot(a[1], W1r[_HD:, :])
    z = jnp.maximum(z + b1r[...], 0.0)
    z = jnp.dot(z, W2r[...]) + b2r[...]
    mu = jnp.mean(z, axis=0, keepdims=True)
    var = jnp.mean((z - mu) * (z - mu), axis=0, keepdims=True)
    zn = (z - mu) * lax.rsqrt(var + 1e-5) * gr[...] + ber[...]
    zn = jnp.maximum(zn, 0.0)
    out[...] = zn

  return pl.pallas_call(
      body,
      out_shape=jax.ShapeDtypeStruct((_N, _D), jnp.float32),
  )(agg, W1, b1.reshape(1, _D), W2, b2.reshape(1, _D),
    g.reshape(1, _D), be.reshape(1, _D))


def kernel(x, edge_index, W1_0, b1_0, W2_0, b2_0, g_0, be_0, W1_1, b1_1,
           W2_1, b2_1, g_1, be_1, W1_2, b1_2, W2_2, b2_2, g_2, be_2):
  x = x.astype(jnp.bfloat16).astype(jnp.float32)
  et = edge_index.reshape(2, _NS, _CH, _C)
  params = [(W1_0, b1_0, W2_0, b2_0, g_0, be_0),
            (W1_1, b1_1, W2_1, b2_1, g_1, be_1),
            (W1_2, b1_2, W2_2, b2_2, g_2, be_2)]
  xs = jnp.stack([x[:, :_HD].reshape(_N // 2, _D),
                  x[:, _HD:].reshape(_N // 2, _D)])
  for l, (W1, b1, W2, b2, g, be) in enumerate(params):
    agg = _sc_aggregate(xs.reshape(_NC, _N, _HD), et)
    last = l == len(params) - 1
    xs = _mlp_bn(agg, W1, b1, W2, b2, g, be, split_out=not last)
  return xs
